# baseline (speedup 1.0000x reference)
import functools

import jax
import jax.numpy as jnp
from jax import lax
from jax.experimental import pallas as pl
from jax.experimental.pallas import tpu as pltpu
from jax.experimental.pallas import tpu_sc as plsc

N_VOCABS = 1000000
EMB_DIM = 64
BATCH = 4096
SEQLEN = 200

NW = 32
BBLK = BATCH // NW
NG = 4
NT = 2
GLA = 2

VBLK = 4096


def _fmt_body(tT_ref, out_ref):
    xt = tT_ref[...].T
    out_ref[...] = jnp.concatenate([xt, xt], axis=-1)


@jax.jit
def _tc_format(tableT):
    grid = (N_VOCABS + VBLK - 1) // VBLK
    return pl.pallas_call(
        _fmt_body,
        grid=(grid,),
        in_specs=[pl.BlockSpec((EMB_DIM, VBLK), lambda i: (0, i))],
        out_specs=pl.BlockSpec((VBLK, 128), lambda i: (i, 0)),
        out_shape=jax.ShapeDtypeStruct((N_VOCABS, 128), jnp.float32),
    )(tableT)


def _emb_body(idxT_hbm, table_hbm, out_hbm, idx_v, rows_v, trans_v, *sems):
    gsem = sems[:NG]
    osem = sems[NG:]
    cid = lax.axis_index("c")
    sid = lax.axis_index("s")
    wid = sid * 2 + cid
    bbase = wid * BBLK

    pltpu.sync_copy(idxT_hbm.at[:, pl.ds(bbase, BBLK)], idx_v)

    iota = lax.iota(jnp.int32, 16)
    d_vecs = [iota + (16 * k) for k in range(4)]

    def start_gather(s, slot):
        pltpu.make_async_copy(
            table_hbm.at[idx_v.at[s]], rows_v.at[slot], gsem[slot]
        ).start()

    def wait_gather(s, slot):
        pltpu.make_async_copy(
            table_hbm.at[idx_v.at[s]], rows_v.at[slot], gsem[slot]
        ).wait()

    def transpose(slot, tslot):
        rv = rows_v.at[slot]
        tv = trans_v.at[tslot]

        @pl.loop(0, BBLK, step=2)
        def _b(b):
            for u in range(2):
                bb = b + u
                bvec = jnp.full((16,), 0, jnp.int32) + bb
                loads = [rv[bb, pl.ds(16 * k, 16)] for k in range(4)]
                for k in range(4):
                    plsc.store_scatter(tv, [d_vecs[k], bvec], loads[k])

    def start_write(s, tslot):
        pltpu.make_async_copy(
            trans_v.at[tslot, :, pl.ds(0, BBLK)],
            out_hbm.at[s, :, pl.ds(bbase, BBLK)],
            osem[tslot],
        ).start()

    def wait_write(s, tslot):
        pltpu.make_async_copy(
            trans_v.at[tslot, :, pl.ds(0, BBLK)],
            out_hbm.at[s, :, pl.ds(bbase, BBLK)],
            osem[tslot],
        ).wait()

    start_gather(0, 0)
    start_gather(1, 1)
    for s in range(2):
        wait_gather(s, s % NG)
        transpose(s % NG, s % NT)
        start_write(s, s % NT)
        start_gather(s + GLA, (s + GLA) % NG)

    @pl.loop(2, SEQLEN - GLA, step=4)
    def _steady(s0):
        for b in range(4):
            s = s0 + b
            sg = (2 + b) % NG
            st = b % NT
            wait_gather(s, sg)
            wait_write(s - NT, st)
            transpose(sg, st)
            start_write(s, st)
            start_gather(s + GLA, b % NG)

    for s in range(SEQLEN - GLA, SEQLEN):
        sg = s % NG
        st = s % NT
        wait_gather(s, sg)
        wait_write(s - NT, st)
        transpose(sg, st)
        start_write(s, st)
    for s in range(SEQLEN - NT, SEQLEN):
        wait_write(s, s % NT)


@jax.jit
def _emb_lookup(idxT, table):
    k = pl.kernel(
        _emb_body,
        out_type=jax.ShapeDtypeStruct((SEQLEN, EMB_DIM, BATCH), jnp.float32),
        mesh=plsc.VectorSubcoreMesh(core_axis_name="c", subcore_axis_name="s"),
        compiler_params=pltpu.CompilerParams(
            use_tc_tiling_on_sc=True, needs_layout_passes=False
        ),
        scratch_types=(
            [
                pltpu.VMEM((SEQLEN, BBLK), jnp.int32),
                pltpu.VMEM((NG, BBLK, 128), jnp.float32),
                pltpu.VMEM((NT, EMB_DIM, BBLK + 1), jnp.float32),
            ]
            + [pltpu.SemaphoreType.DMA] * (NG + NT)
        ),
    )
    return k(idxT, table)


def kernel(input, emb_weight):
    table_pad = _tc_format(emb_weight.T)
    out = _emb_lookup(input.T, table_pad)
    return out.transpose(2, 0, 1)


# speedup vs baseline: 1.7579x; 1.7579x over previous
import functools

import jax
import jax.numpy as jnp
from jax import lax
from jax.experimental import pallas as pl
from jax.experimental.pallas import tpu as pltpu
from jax.experimental.pallas import tpu_sc as plsc

N_VOCABS = 1000000
EMB_DIM = 64
BATCH = 4096
SEQLEN = 200

CHUNK = 128
TOTAL = BATCH * SEQLEN
NROWS = TOTAL // CHUNK
NW = 32
ROWS_PER_W = NROWS // NW
NBUF = 5
G = 3

VBLK = 8192


def _fmt_body(tT_ref, out_ref):
    xt = tT_ref[...].T
    out_ref[...] = jnp.concatenate([xt, xt], axis=-1)


@jax.jit
def _tc_format(tableT):
    grid = (N_VOCABS + VBLK - 1) // VBLK
    return pl.pallas_call(
        _fmt_body,
        grid=(grid,),
        in_specs=[pl.BlockSpec((EMB_DIM, VBLK), lambda i: (0, i))],
        out_specs=pl.BlockSpec((VBLK, 128), lambda i: (i, 0)),
        out_shape=jax.ShapeDtypeStruct((N_VOCABS, 128), jnp.float32),
    )(tableT)


def _emb_body(idx_hbm, table_hbm, out_hbm, idx_v, rows_v, *sems):
    gsem = sems[:NBUF]
    osem = sems[NBUF:]
    cid = lax.axis_index("c")
    sid = lax.axis_index("s")
    wid = sid * 2 + cid
    base = wid * ROWS_PER_W

    pltpu.sync_copy(idx_hbm.at[pl.ds(base, ROWS_PER_W)], idx_v)

    def start_gather(j, slot):
        pltpu.make_async_copy(
            table_hbm.at[idx_v.at[j]], rows_v.at[slot], gsem[slot]
        ).start()

    def wait_gather(j, slot):
        pltpu.make_async_copy(
            table_hbm.at[idx_v.at[j]], rows_v.at[slot], gsem[slot]
        ).wait()

    def start_write(j, slot):
        pltpu.make_async_copy(
            rows_v.at[slot], out_hbm.at[base + j], osem[slot]
        ).start()

    def wait_write(j, slot):
        pltpu.make_async_copy(
            rows_v.at[slot], out_hbm.at[base + j], osem[slot]
        ).wait()

    for j in range(G):
        start_gather(j, j % NBUF)
    for j in range(NBUF - G):
        wait_gather(j, j % NBUF)
        start_write(j, j % NBUF)
        start_gather(j + G, (j + G) % NBUF)

    @pl.loop(NBUF - G, ROWS_PER_W - G, step=NBUF)
    def _steady(j0):
        for b in range(NBUF):
            j = j0 + b
            slot = (NBUF - G + b) % NBUF
            sn = (NBUF - G + b + G) % NBUF
            wait_gather(j, slot)
            start_write(j, slot)
            wait_write(j + G - NBUF, sn)
            start_gather(j + G, sn)

    for j in range(ROWS_PER_W - G, ROWS_PER_W):
        wait_gather(j, j % NBUF)
        start_write(j, j % NBUF)
    for j in range(ROWS_PER_W - NBUF, ROWS_PER_W):
        wait_write(j, j % NBUF)


@jax.jit
def _emb_lookup(idx, table):
    k = pl.kernel(
        _emb_body,
        out_type=jax.ShapeDtypeStruct((NROWS, CHUNK, 128), jnp.float32),
        mesh=plsc.VectorSubcoreMesh(core_axis_name="c", subcore_axis_name="s"),
        compiler_params=pltpu.CompilerParams(
            use_tc_tiling_on_sc=True, needs_layout_passes=False
        ),
        scratch_types=(
            [
                pltpu.VMEM((ROWS_PER_W, CHUNK), jnp.int32),
                pltpu.VMEM((NBUF, CHUNK, 128), jnp.float32),
            ]
            + [pltpu.SemaphoreType.DMA] * (2 * NBUF)
        ),
    )
    return k(idx, table)


def kernel(input, emb_weight):
    table_pad = _tc_format(emb_weight.T)
    idx = input.reshape(NROWS, CHUNK)
    out = _emb_lookup(idx, table_pad)
    return out[:, :, :EMB_DIM].reshape(BATCH, SEQLEN, EMB_DIM)
